# speedup vs baseline: 3.8286x; 3.8286x over previous
import functools

import jax
import jax.numpy as jnp
from jax import lax
from jax.experimental import pallas as pl
from jax.experimental.pallas import tpu as pltpu
from jax.experimental.pallas import tpu_sc as plsc

NSEG = 1024
HID = 128
NROWS = 320000
NC = 2
NS = 16
NW = NC * NS
RPT = NROWS // NW
CHUNK = 80
NCHUNK = RPT // CHUNK
SEG_PER_TILE = NSEG // NS


def _seg_sum_sc(x, batch, zeros):
    mesh = plsc.VectorSubcoreMesh(core_axis_name="c", subcore_axis_name="s")

    @functools.partial(
        pl.kernel,
        out_type=jax.ShapeDtypeStruct((NC * NSEG, HID), jnp.float32),
        mesh=mesh,
        scratch_types=[
            pltpu.VMEM((CHUNK, HID), jnp.float32),
            pltpu.VMEM((CHUNK,), jnp.int32),
            pltpu.VMEM_SHARED((NSEG, HID), jnp.float32),
        ],
    )
    def k(x_hbm, b_hbm, z_hbm, out_hbm, xbuf, idbuf, acc):
        cid = lax.axis_index("c")
        sid = lax.axis_index("s")
        wid = sid * NC + cid
        pltpu.sync_copy(z_hbm, acc.at[pl.ds(sid * SEG_PER_TILE, SEG_PER_TILE)])
        plsc.subcore_barrier()

        base = wid * RPT

        def chunk_body(kk, _):
            r0 = base + kk * CHUNK
            pltpu.sync_copy(x_hbm.at[pl.ds(r0, CHUNK)], xbuf)
            pltpu.sync_copy(b_hbm.at[pl.ds(r0, CHUNK)], idbuf)
            pltpu.sync_copy(xbuf, acc.at[idbuf], add=True)
            return 0

        lax.fori_loop(0, NCHUNK, chunk_body, 0)
        plsc.subcore_barrier()
        pltpu.sync_copy(
            acc.at[pl.ds(sid * SEG_PER_TILE, SEG_PER_TILE)],
            out_hbm.at[pl.ds(cid * NSEG + sid * SEG_PER_TILE, SEG_PER_TILE)],
        )

    return k(x, batch, zeros)


def _mlp_body(p_ref, w1_ref, b1_ref, g_ref, be_ref, w2_ref, b2_ref, o_ref):
    pooled = p_ref[:NSEG, :] + p_ref[NSEG:, :]
    h = jnp.dot(pooled, w1_ref[...], preferred_element_type=jnp.float32,
                precision=lax.Precision.HIGHEST)
    h = h + b1_ref[...]
    mean = jnp.mean(h, axis=-1, keepdims=True)
    var = jnp.mean((h - mean) ** 2, axis=-1, keepdims=True)
    h = (h - mean) * lax.rsqrt(var + 1e-5) * g_ref[...] + be_ref[...]
    h = jnp.maximum(h, 0.0)
    o_ref[...] = jnp.dot(h, w2_ref[...], preferred_element_type=jnp.float32,
                         precision=lax.Precision.HIGHEST) + b2_ref[0, 0]


def _mlp_tc(pooled2, W1, b1, gamma, beta, W2, b2):
    return pl.pallas_call(
        _mlp_body,
        out_shape=jax.ShapeDtypeStruct((NSEG, 1), jnp.float32),
    )(pooled2, W1, b1.reshape(1, HID), gamma.reshape(1, HID),
      beta.reshape(1, HID), W2, b2.reshape(1, 1))


def kernel(x, batch, W1, b1, gamma, beta, W2, b2):
    batch = batch.astype(jnp.int32)
    zeros = jnp.zeros((SEG_PER_TILE, HID), jnp.float32)
    pooled2 = _seg_sum_sc(x, batch, zeros)
    return _mlp_tc(pooled2, W1, b1, gamma, beta, W2, b2)
